# baseline (speedup 1.0000x reference)
import functools

import jax
import jax.numpy as jnp
from jax import lax
from jax.experimental import pallas as pl
from jax.experimental.pallas import tpu as pltpu
from jax.experimental.pallas import tpu_sc as plsc

_IDX_ROW = 128
_CH_ROWS = 16
_CHUNK = _IDX_ROW * _CH_ROWS
_NW = 32


@jax.jit
def _embed_lookup(x, table):
    b_total = x.shape[0]
    n_rows, d = table.shape
    dhi = d // 8
    n_jblk = b_total // _IDX_ROW
    b_per_w = b_total // _NW
    steps = b_per_w // _CHUNK
    assert steps % 2 == 0
    half = steps // 2

    mesh = plsc.VectorSubcoreMesh(core_axis_name="c", subcore_axis_name="s")

    @functools.partial(
        pl.kernel,
        mesh=mesh,
        out_type=jax.ShapeDtypeStruct((dhi, n_jblk, 8, _IDX_ROW), jnp.float32),
        scratch_types=[
            pltpu.VMEM((_CHUNK,), jnp.int32),
            pltpu.VMEM((_CHUNK,), jnp.int32),
            pltpu.VMEM((_CH_ROWS, _IDX_ROW, d), jnp.float32),
            pltpu.VMEM((_CH_ROWS, _IDX_ROW, d), jnp.float32),
            pltpu.SemaphoreType.DMA,
            pltpu.SemaphoreType.DMA,
            pltpu.SemaphoreType.DMA,
            pltpu.SemaphoreType.DMA,
        ],
        compiler_params=pltpu.CompilerParams(use_tc_tiling_on_sc=False),
    )
    def k(x_hbm, table_hbm, out_hbm, idx0, idx1, buf0, buf1,
          sg0, sg1, so0, so1):
        wid = lax.axis_index("s") * 2 + lax.axis_index("c")
        row0 = wid * b_per_w
        jblk0 = wid * (b_per_w // _IDX_ROW)

        idx_bufs = (idx0, idx1)
        row_bufs = (buf0, buf1)
        g_sems = (sg0, sg1)
        o_sems = (so0, so1)

        def load_idx(s, p):
            pltpu.sync_copy(
                x_hbm.at[pl.ds(row0 + s * _CHUNK, _CHUNK)], idx_bufs[p])

        def fire_gathers(p):
            for j in range(_CH_ROWS):
                pltpu.async_copy(
                    table_hbm.at[idx_bufs[p].at[pl.ds(j * _IDX_ROW, _IDX_ROW)]],
                    row_bufs[p].at[j],
                    g_sems[p])

        def drain_gathers(p):
            for j in range(_CH_ROWS):
                pltpu.make_async_copy(
                    table_hbm.at[idx_bufs[p].at[pl.ds(j * _IDX_ROW, _IDX_ROW)]],
                    row_bufs[p].at[j],
                    g_sems[p]).wait()

        def fire_stores(s, p):
            jb = jblk0 + s * _CH_ROWS
            for i in range(dhi):
                for a in range(8):
                    pltpu.async_copy(
                        row_bufs[p].at[:, :, 8 * i + a],
                        out_hbm.at[i, pl.ds(jb, _CH_ROWS), a],
                        o_sems[p])

        def drain_stores(s, p):
            jb = jblk0 + s * _CH_ROWS
            for i in range(dhi):
                for a in range(8):
                    pltpu.make_async_copy(
                        row_bufs[p].at[:, :, 8 * i + a],
                        out_hbm.at[i, pl.ds(jb, _CH_ROWS), a],
                        o_sems[p]).wait()

        load_idx(0, 0)
        fire_gathers(0)

        def body(g, carry):
            s0 = g * 2
            load_idx(s0 + 1, 1)
            drain_gathers(0)
            fire_stores(s0, 0)

            @pl.when(g > 0)
            def _():
                drain_stores(s0 - 1, 1)
            fire_gathers(1)

            @pl.when(g < half - 1)
            def _():
                load_idx(s0 + 2, 0)
            drain_gathers(1)
            fire_stores(s0 + 1, 1)
            drain_stores(s0, 0)

            @pl.when(g < half - 1)
            def _():
                fire_gathers(0)
            return carry

        lax.fori_loop(0, half, body, 0)
        drain_stores(steps - 1, 1)

    out4d = k(x, table)
    return out4d.transpose(1, 3, 0, 2).reshape(b_total, d)


def kernel(x, table):
    return _embed_lookup(x.astype(jnp.int32), table)


# speedup vs baseline: 89.9539x; 89.9539x over previous
import functools

import jax
import jax.numpy as jnp
from jax import lax
from jax.experimental import pallas as pl
from jax.experimental.pallas import tpu as pltpu
from jax.experimental.pallas import tpu_sc as plsc

_IDX_ROW = 128
_CH_ROWS = 8
_CHUNK = _IDX_ROW * _CH_ROWS
_NW = 32
_L = 16
_TILE = 1024


@jax.jit
def _embed_lookup(x, table):
    b_total = x.shape[0]
    d = table.shape[1]
    dhi = d // 8
    n_jblk = b_total // _IDX_ROW
    b_per_w = b_total // _NW
    steps = b_per_w // _CHUNK
    assert steps % 2 == 0
    half = steps // 2
    ob_words = _CH_ROWS * 8 * _IDX_ROW

    mesh = plsc.VectorSubcoreMesh(core_axis_name="c", subcore_axis_name="s")

    @functools.partial(
        pl.kernel,
        mesh=mesh,
        out_type=jax.ShapeDtypeStruct((dhi, n_jblk * _TILE), jnp.float32),
        scratch_types=[
            pltpu.VMEM((_CHUNK,), jnp.int32),
            pltpu.VMEM((_CHUNK,), jnp.int32),
            pltpu.VMEM((_CHUNK, d), jnp.float32),
            pltpu.VMEM((_CHUNK, d), jnp.float32),
            pltpu.VMEM((dhi, ob_words), jnp.float32),
            pltpu.VMEM((dhi, ob_words), jnp.float32),
            pltpu.SemaphoreType.DMA,
            pltpu.SemaphoreType.DMA,
            pltpu.SemaphoreType.DMA,
            pltpu.SemaphoreType.DMA,
        ],
        compiler_params=pltpu.CompilerParams(
            use_tc_tiling_on_sc=False, needs_layout_passes=False),
    )
    def k(x_hbm, table_hbm, out_hbm, idx0, idx1, rows0, rows1, ob0, ob1,
          sg0, sg1, so0, so1):
        wid = lax.axis_index("s") * 2 + lax.axis_index("c")
        row0 = wid * b_per_w
        jblk0 = wid * (b_per_w // _IDX_ROW)

        idx_bufs = (idx0, idx1)
        row_bufs = (rows0, rows1)
        o_bufs = (ob0, ob1)
        g_sems = (sg0, sg1)
        o_sems = (so0, so1)

        def load_idx(s, p):
            pltpu.sync_copy(
                x_hbm.at[pl.ds(row0 + s * _CHUNK, _CHUNK)], idx_bufs[p])

        def fire_gathers(p):
            for j in range(_CH_ROWS):
                pltpu.async_copy(
                    table_hbm.at[idx_bufs[p].at[pl.ds(j * _IDX_ROW, _IDX_ROW)]],
                    row_bufs[p].at[pl.ds(j * _IDX_ROW, _IDX_ROW)],
                    g_sems[p])

        def drain_gathers(p):
            for j in range(_CH_ROWS):
                pltpu.make_async_copy(
                    table_hbm.at[idx_bufs[p].at[pl.ds(j * _IDX_ROW, _IDX_ROW)]],
                    row_bufs[p].at[pl.ds(j * _IDX_ROW, _IDX_ROW)],
                    g_sems[p]).wait()

        def transpose_chunk(p):
            rows2 = row_bufs[p]
            ob = o_bufs[p]
            lanes = lax.iota(jnp.int32, _L)

            def jloop(jj, carry):
                rbase = jj * _IDX_ROW
                obase = jj * _TILE
                for b0 in range(_IDX_ROW // _L):
                    rvec = lanes + (rbase + b0 * _L)
                    for i in range(dhi):
                        for a in range(8):
                            cv = jnp.full((_L,), 8 * i + a, jnp.int32)
                            v = plsc.load_gather(rows2, [rvec, cv])
                            ob[i, pl.ds(obase + a * _IDX_ROW + b0 * _L, _L)] = v
                return carry

            lax.fori_loop(0, _CH_ROWS, jloop, 0)

        def fire_stores(s, p):
            w0 = (jblk0 + s * _CH_ROWS) * _TILE
            for i in range(dhi):
                pltpu.async_copy(
                    o_bufs[p].at[i],
                    out_hbm.at[i, pl.ds(w0, ob_words)],
                    o_sems[p])

        def drain_stores(s, p):
            w0 = (jblk0 + s * _CH_ROWS) * _TILE
            for i in range(dhi):
                pltpu.make_async_copy(
                    o_bufs[p].at[i],
                    out_hbm.at[i, pl.ds(w0, ob_words)],
                    o_sems[p]).wait()

        load_idx(0, 0)
        fire_gathers(0)

        def body(g, carry):
            s0 = g * 2
            load_idx(s0 + 1, 1)
            drain_gathers(0)
            fire_gathers(1)

            @pl.when(g > 0)
            def _():
                drain_stores(s0 - 2, 0)
            transpose_chunk(0)
            fire_stores(s0, 0)

            @pl.when(g < half - 1)
            def _():
                load_idx(s0 + 2, 0)
            drain_gathers(1)

            @pl.when(g < half - 1)
            def _():
                fire_gathers(0)

            @pl.when(g > 0)
            def _():
                drain_stores(s0 - 1, 1)
            transpose_chunk(1)
            fire_stores(s0 + 1, 1)
            return carry

        lax.fori_loop(0, half, body, 0)
        drain_stores(steps - 2, 0)
        drain_stores(steps - 1, 1)

    out2d = k(x, table)
    out4d = out2d.reshape(dhi, n_jblk, 8, _IDX_ROW)
    return out4d.transpose(1, 3, 0, 2).reshape(b_total, d)


def kernel(x, table):
    return _embed_lookup(x.astype(jnp.int32), table)


# speedup vs baseline: 130.6505x; 1.4524x over previous
import functools

import jax
import jax.numpy as jnp
from jax import lax
from jax.experimental import pallas as pl
from jax.experimental.pallas import tpu as pltpu
from jax.experimental.pallas import tpu_sc as plsc

_IDX_ROW = 128
_CH_ROWS = 8
_CHUNK = _IDX_ROW * _CH_ROWS
_NW = 32
_L = 16
_TILE = 1024


@jax.jit
def _embed_lookup(x, table):
    b_total = x.shape[0]
    d = table.shape[1]
    dhi = d // 8
    n_jblk = b_total // _IDX_ROW
    b_per_w = b_total // _NW
    steps = b_per_w // _CHUNK
    assert steps % 2 == 0
    half = steps // 2
    ob_words = _CH_ROWS * 8 * _IDX_ROW

    mesh = plsc.VectorSubcoreMesh(core_axis_name="c", subcore_axis_name="s")

    @functools.partial(
        pl.kernel,
        mesh=mesh,
        out_type=jax.ShapeDtypeStruct((dhi, n_jblk * _TILE), jnp.float32),
        scratch_types=[
            pltpu.VMEM((_CHUNK,), jnp.int32),
            pltpu.VMEM((_CHUNK,), jnp.int32),
            pltpu.VMEM((_CHUNK, d), jnp.float32),
            pltpu.VMEM((_CHUNK, d), jnp.float32),
            pltpu.VMEM((dhi, ob_words), jnp.float32),
            pltpu.VMEM((dhi, ob_words), jnp.float32),
            pltpu.SemaphoreType.DMA,
            pltpu.SemaphoreType.DMA,
            pltpu.SemaphoreType.DMA,
            pltpu.SemaphoreType.DMA,
        ],
        compiler_params=pltpu.CompilerParams(
            use_tc_tiling_on_sc=False, needs_layout_passes=False),
    )
    def k(x_hbm, table_hbm, out_hbm, idx0, idx1, rows0, rows1, ob0, ob1,
          sg0, sg1, so0, so1):
        wid = lax.axis_index("s") * 2 + lax.axis_index("c")
        row0 = wid * b_per_w
        jblk0 = wid * (b_per_w // _IDX_ROW)

        idx_bufs = (idx0, idx1)
        row_bufs = (rows0, rows1)
        o_bufs = (ob0, ob1)
        g_sems = (sg0, sg1)
        o_sems = (so0, so1)

        def load_idx(s, p):
            pltpu.sync_copy(
                x_hbm.at[pl.ds(row0 + s * _CHUNK, _CHUNK)], idx_bufs[p])

        def fire_gathers(p):
            for j in range(_CH_ROWS):
                pltpu.async_copy(
                    table_hbm.at[idx_bufs[p].at[pl.ds(j * _IDX_ROW, _IDX_ROW)]],
                    row_bufs[p].at[pl.ds(j * _IDX_ROW, _IDX_ROW)],
                    g_sems[p])

        def drain_gathers(p):
            for j in range(_CH_ROWS):
                pltpu.make_async_copy(
                    table_hbm.at[idx_bufs[p].at[pl.ds(j * _IDX_ROW, _IDX_ROW)]],
                    row_bufs[p].at[pl.ds(j * _IDX_ROW, _IDX_ROW)],
                    g_sems[p]).wait()

        def transpose_chunk(p):
            rows2 = row_bufs[p]
            ob = o_bufs[p]
            lanes = lax.iota(jnp.int32, _L)

            def jloop(jj, carry):
                rbase = jj * _IDX_ROW
                obase = jj * _TILE
                for b0 in range(_IDX_ROW // _L):
                    rvec = lanes + (rbase + b0 * _L)
                    for c in range(d):
                        cvec = (lanes + c) % d
                        v = plsc.load_gather(rows2, [rvec, cvec])
                        ivec = cvec // 8
                        svec = (cvec % 8) * _IDX_ROW + lanes + (obase + b0 * _L)
                        plsc.store_scatter(ob, [ivec, svec], v)
                return carry

            lax.fori_loop(0, _CH_ROWS, jloop, 0)

        def fire_stores(s, p):
            w0 = (jblk0 + s * _CH_ROWS) * _TILE
            for i in range(dhi):
                pltpu.async_copy(
                    o_bufs[p].at[i],
                    out_hbm.at[i, pl.ds(w0, ob_words)],
                    o_sems[p])

        def drain_stores(s, p):
            w0 = (jblk0 + s * _CH_ROWS) * _TILE
            for i in range(dhi):
                pltpu.make_async_copy(
                    o_bufs[p].at[i],
                    out_hbm.at[i, pl.ds(w0, ob_words)],
                    o_sems[p]).wait()

        load_idx(0, 0)
        fire_gathers(0)

        def body(g, carry):
            s0 = g * 2
            load_idx(s0 + 1, 1)
            drain_gathers(0)
            fire_gathers(1)

            @pl.when(g > 0)
            def _():
                drain_stores(s0 - 2, 0)
            transpose_chunk(0)
            fire_stores(s0, 0)

            @pl.when(g < half - 1)
            def _():
                load_idx(s0 + 2, 0)
            drain_gathers(1)

            @pl.when(g < half - 1)
            def _():
                fire_gathers(0)

            @pl.when(g > 0)
            def _():
                drain_stores(s0 - 1, 1)
            transpose_chunk(1)
            fire_stores(s0 + 1, 1)
            return carry

        lax.fori_loop(0, half, body, 0)
        drain_stores(steps - 2, 0)
        drain_stores(steps - 1, 1)

    out2d = k(x, table)
    out4d = out2d.reshape(dhi, n_jblk, 8, _IDX_ROW)
    return out4d.transpose(1, 3, 0, 2).reshape(b_total, d)


def kernel(x, table):
    return _embed_lookup(x.astype(jnp.int32), table)


# speedup vs baseline: 142.6528x; 1.0919x over previous
import functools

import jax
import jax.numpy as jnp
from jax import lax
from jax.experimental import pallas as pl
from jax.experimental.pallas import tpu as pltpu
from jax.experimental.pallas import tpu_sc as plsc

_IDX_ROW = 128
_CH_ROWS = 10
_CHUNK = _IDX_ROW * _CH_ROWS
_NW = 32
_L = 16
_TILE = 1024


@jax.jit
def _embed_lookup(x, table):
    b_total = x.shape[0]
    d = table.shape[1]
    dhi = d // 8
    n_jblk = b_total // _IDX_ROW
    b_per_w = b_total // _NW
    steps = b_per_w // _CHUNK
    assert steps % 2 == 0
    half = steps // 2
    ob_words = _CH_ROWS * 8 * _IDX_ROW

    mesh = plsc.VectorSubcoreMesh(core_axis_name="c", subcore_axis_name="s")

    @functools.partial(
        pl.kernel,
        mesh=mesh,
        out_type=jax.ShapeDtypeStruct((dhi, n_jblk * _TILE), jnp.float32),
        scratch_types=[
            pltpu.VMEM((_CHUNK,), jnp.int32),
            pltpu.VMEM((_CHUNK,), jnp.int32),
            pltpu.VMEM((_CHUNK, d), jnp.float32),
            pltpu.VMEM((_CHUNK, d), jnp.float32),
            pltpu.VMEM((dhi, ob_words), jnp.float32),
            pltpu.VMEM((dhi, ob_words), jnp.float32),
            pltpu.SemaphoreType.DMA,
            pltpu.SemaphoreType.DMA,
            pltpu.SemaphoreType.DMA,
            pltpu.SemaphoreType.DMA,
            pltpu.SemaphoreType.DMA,
            pltpu.SemaphoreType.DMA,
        ],
        compiler_params=pltpu.CompilerParams(
            use_tc_tiling_on_sc=False, needs_layout_passes=False),
    )
    def k(x_hbm, table_hbm, out_hbm, idx0, idx1, rows0, rows1, ob0, ob1,
          sg0, sg1, so0, so1, si0, si1):
        wid = lax.axis_index("s") * 2 + lax.axis_index("c")
        row0 = wid * b_per_w
        jblk0 = wid * (b_per_w // _IDX_ROW)

        idx_bufs = (idx0, idx1)
        row_bufs = (rows0, rows1)
        o_bufs = (ob0, ob1)
        g_sems = (sg0, sg1)
        o_sems = (so0, so1)
        i_sems = (si0, si1)

        def fire_idx(s, p):
            pltpu.async_copy(
                x_hbm.at[pl.ds(row0 + s * _CHUNK, _CHUNK)], idx_bufs[p],
                i_sems[p])

        def wait_idx(s, p):
            pltpu.make_async_copy(
                x_hbm.at[pl.ds(row0 + s * _CHUNK, _CHUNK)], idx_bufs[p],
                i_sems[p]).wait()

        def fire_gathers(p):
            for j in range(_CH_ROWS):
                pltpu.async_copy(
                    table_hbm.at[idx_bufs[p].at[pl.ds(j * _IDX_ROW, _IDX_ROW)]],
                    row_bufs[p].at[pl.ds(j * _IDX_ROW, _IDX_ROW)],
                    g_sems[p])

        def drain_gathers(p):
            for j in range(_CH_ROWS):
                pltpu.make_async_copy(
                    table_hbm.at[idx_bufs[p].at[pl.ds(j * _IDX_ROW, _IDX_ROW)]],
                    row_bufs[p].at[pl.ds(j * _IDX_ROW, _IDX_ROW)],
                    g_sems[p]).wait()

        def transpose_chunk(p):
            rows2 = row_bufs[p]
            ob = o_bufs[p]
            lanes = lax.iota(jnp.int32, _L)

            def jloop(jj, carry):
                rbase = jj * _IDX_ROW
                obase = jj * _TILE
                for b0 in range(_IDX_ROW // _L):
                    rvec = lanes + (rbase + b0 * _L)
                    for c in range(d):
                        cpl = lanes + c
                        cvec = cpl & (d - 1)
                        v = plsc.load_gather(rows2, [rvec, cvec])
                        ivec = (cpl >> 3) & 1
                        svec = ((cpl & 7) << 7) + lanes + (obase + b0 * _L)
                        plsc.store_scatter(ob, [ivec, svec], v)
                return carry

            lax.fori_loop(0, _CH_ROWS, jloop, 0)

        def fire_stores(s, p):
            w0 = (jblk0 + s * _CH_ROWS) * _TILE
            for i in range(dhi):
                pltpu.async_copy(
                    o_bufs[p].at[i],
                    out_hbm.at[i, pl.ds(w0, ob_words)],
                    o_sems[p])

        def drain_stores(s, p):
            w0 = (jblk0 + s * _CH_ROWS) * _TILE
            for i in range(dhi):
                pltpu.make_async_copy(
                    o_bufs[p].at[i],
                    out_hbm.at[i, pl.ds(w0, ob_words)],
                    o_sems[p]).wait()

        fire_idx(0, 0)
        wait_idx(0, 0)
        fire_gathers(0)
        fire_idx(1, 1)

        def body(g, carry):
            s0 = g * 2
            wait_idx(s0 + 1, 1)
            drain_gathers(0)
            fire_gathers(1)

            @pl.when(g < half - 1)
            def _():
                fire_idx(s0 + 2, 0)

            @pl.when(g > 0)
            def _():
                drain_stores(s0 - 2, 0)
            transpose_chunk(0)
            fire_stores(s0, 0)

            @pl.when(g < half - 1)
            def _():
                wait_idx(s0 + 2, 0)
            drain_gathers(1)

            @pl.when(g < half - 1)
            def _():
                fire_gathers(0)
                fire_idx(s0 + 3, 1)

            @pl.when(g > 0)
            def _():
                drain_stores(s0 - 1, 1)
            transpose_chunk(1)
            fire_stores(s0 + 1, 1)
            return carry

        lax.fori_loop(0, half, body, 0)
        drain_stores(steps - 2, 0)
        drain_stores(steps - 1, 1)

    out2d = k(x, table)
    out4d = out2d.reshape(dhi, n_jblk, 8, _IDX_ROW)
    return out4d.transpose(1, 3, 0, 2).reshape(b_total, d)


def kernel(x, table):
    return _embed_lookup(x.astype(jnp.int32), table)
